# baseline (speedup 1.0000x reference)
import functools

import jax
import jax.numpy as jnp
from jax import lax
from jax.experimental import pallas as pl
from jax.experimental.pallas import tpu as pltpu
from jax.experimental.pallas import tpu_sc as plsc

N = 10000
E = 320000
D = 128

NC = 2
NS = 16
NW = NC * NS

CHUNK = 64
EDGES_PER_W = E // NW
NCHUNK = 157
EDGES_PAD_W = NCHUNK * CHUNK
NPAD = 10112
ROWS_PER_TILE = NPAD // NS

@functools.cache
def _make_sc_aggregate():
    mesh = plsc.VectorSubcoreMesh(
        core_axis_name="c", subcore_axis_name="s", num_cores=NC,
        num_subcores=NS)
    return pl.kernel(
        _sc_aggregate_body,
        out_type=jax.ShapeDtypeStruct((NC, NPAD, D), jnp.float32),
        mesh=mesh,
        scratch_types=[
            pltpu.VMEM((EDGES_PAD_W,), jnp.int32),
            pltpu.VMEM((NCHUNK, CHUNK), jnp.int32),
            pltpu.VMEM((CHUNK, D), jnp.float32),
            pltpu.VMEM((CHUNK, D), jnp.float32),
            pltpu.VMEM_SHARED((NPAD, D), jnp.float32),
            pltpu.SemaphoreType.DMA,
            pltpu.SemaphoreType.DMA,
            pltpu.SemaphoreType.DMA,
            pltpu.SemaphoreType.DMA,
        ],
    )


def _sc_aggregate_body(x_hbm, src_hbm, dst_hbm, zero_hbm, out_hbm,
                       src_v, dst_v, rows0, rows1, agg_sh,
                       gsem0, gsem1, ssem0, ssem1):
    c = lax.axis_index("c")
    s = lax.axis_index("s")
    wid = s * NC + c

    pltpu.sync_copy(zero_hbm.at[pl.ds(s * ROWS_PER_TILE, ROWS_PER_TILE)],
                    agg_sh.at[pl.ds(s * ROWS_PER_TILE, ROWS_PER_TILE)])
    pltpu.sync_copy(src_hbm.at[wid], src_v)
    pltpu.sync_copy(dst_hbm.at[wid], dst_v)
    plsc.subcore_barrier()

    def gidx(i):
        return src_v.at[pl.ds(pl.multiple_of(i * CHUNK, 8), CHUNK)]

    pltpu.async_copy(x_hbm.at[gidx(0)], rows0, gsem0)

    def pair(t, carry):
        i0 = 2 * t
        i1 = i0 + 1
        pltpu.make_async_copy(x_hbm.at[gidx(i0)], rows0, gsem0).wait()

        @pl.when(t > 0)
        def _():
            pltpu.make_async_copy(rows1, agg_sh.at[dst_v.at[i1]],
                                  ssem1).wait()

        pltpu.async_copy(x_hbm.at[gidx(i1)], rows1, gsem1)
        pltpu.async_copy(rows0, agg_sh.at[dst_v.at[i0]], ssem0, add=True)
        pltpu.make_async_copy(x_hbm.at[gidx(i1)], rows1, gsem1).wait()
        pltpu.make_async_copy(rows0, agg_sh.at[dst_v.at[i0]], ssem0).wait()

        @pl.when(i0 + 2 < NCHUNK)
        def _():
            pltpu.async_copy(x_hbm.at[gidx(i0 + 2)], rows0, gsem0)

        pltpu.async_copy(rows1, agg_sh.at[dst_v.at[i1]], ssem1, add=True)
        return carry

    lax.fori_loop(0, NCHUNK // 2, pair, 0)
    if NCHUNK % 2:
        pltpu.make_async_copy(x_hbm.at[gidx(NCHUNK - 1)], rows0,
                              gsem0).wait()
        pltpu.sync_copy(rows0, agg_sh.at[dst_v.at[NCHUNK - 1]], add=True)
        pltpu.make_async_copy(rows1, agg_sh.at[dst_v.at[NCHUNK - 2]],
                              ssem1).wait()
    else:
        pltpu.make_async_copy(rows1, agg_sh.at[dst_v.at[NCHUNK - 1]],
                              ssem1).wait()
    plsc.subcore_barrier()

    pltpu.sync_copy(agg_sh.at[pl.ds(s * ROWS_PER_TILE, ROWS_PER_TILE)],
                    out_hbm.at[c, pl.ds(s * ROWS_PER_TILE, ROWS_PER_TILE)])


BN = 2000
NB = N // BN


def _tc_body(eps_sm, x_b, a0_b, a1_b, W1_b, b1_b, W2_b, b2_b, g_b, be_b,
             out_b, h2_sc, sum_sc, sq_sc, scale_sc, off_sc):
    p = pl.program_id(0)
    i = pl.program_id(1)

    @pl.when(p == 0)
    def _():
        m = (1.0 + eps_sm[0]) * x_b[...] + a0_b[0] + a1_b[0]
        h1 = jnp.maximum(
            jnp.dot(m, W1_b[...], preferred_element_type=jnp.float32) + b1_b[...],
            0.0)
        h2 = jnp.dot(h1, W2_b[...], preferred_element_type=jnp.float32) + b2_b[...]

        @pl.when(i == 0)
        def _():
            sum_sc[...] = jnp.zeros_like(sum_sc)
            sq_sc[...] = jnp.zeros_like(sq_sc)

        sum_sc[...] += jnp.sum(h2, axis=0, keepdims=True)
        sq_sc[...] += jnp.sum(h2 * h2, axis=0, keepdims=True)
        h2_sc[pl.ds(i * BN, BN), :] = h2
        out_b[...] = h2

    @pl.when(p == 1)
    def _():
        @pl.when(i == 0)
        def _():
            mean = sum_sc[...] * (1.0 / N)
            var = sq_sc[...] * (1.0 / N) - mean * mean
            sc = lax.rsqrt(var + 1e-5) * g_b[...]
            scale_sc[...] = sc
            off_sc[...] = be_b[...] - mean * sc

        h2 = h2_sc[pl.ds(i * BN, BN), :]
        out_b[...] = jnp.maximum(h2 * scale_sc[...] + off_sc[...], 0.0)


def _row_map(p, i):
    return (jnp.where(p == 0, i, 0), 0)


def _tc_mlp_bn(x, agg2, W1, b1, W2, b2, eps, gamma, beta):
    vec = lambda v: v.reshape(1, D)
    a_map = lambda core: (lambda p, i: (core, jnp.where(p == 0, i, 0), 0))
    return pl.pallas_call(
        _tc_body,
        grid=(2, NB),
        in_specs=[
            pl.BlockSpec(memory_space=pltpu.SMEM),
            pl.BlockSpec((BN, D), _row_map),
            pl.BlockSpec((1, BN, D), a_map(0)),
            pl.BlockSpec((1, BN, D), a_map(1)),
            pl.BlockSpec((D, D), lambda p, i: (0, 0)),
            pl.BlockSpec((1, D), lambda p, i: (0, 0)),
            pl.BlockSpec((D, D), lambda p, i: (0, 0)),
            pl.BlockSpec((1, D), lambda p, i: (0, 0)),
            pl.BlockSpec((1, D), lambda p, i: (0, 0)),
            pl.BlockSpec((1, D), lambda p, i: (0, 0)),
        ],
        out_specs=pl.BlockSpec((BN, D), lambda p, i: (i, 0)),
        out_shape=jax.ShapeDtypeStruct((N, D), jnp.float32),
        scratch_shapes=[
            pltpu.VMEM((N, D), jnp.float32),
            pltpu.VMEM((1, D), jnp.float32),
            pltpu.VMEM((1, D), jnp.float32),
            pltpu.VMEM((1, D), jnp.float32),
            pltpu.VMEM((1, D), jnp.float32),
        ],
    )(eps.reshape(1), x, agg2, agg2, W1, vec(b1), W2, vec(b2), vec(gamma),
      vec(beta))


def kernel(x, edge_index, W1, b1, W2, b2, eps, gamma, beta):
    npad_e = EDGES_PAD_W - EDGES_PER_W
    ei = edge_index.reshape(2, NW, EDGES_PER_W)
    if npad_e:
        src = jnp.concatenate(
            [ei[0], jnp.zeros((NW, npad_e), jnp.int32)], axis=1)
        dst = jnp.concatenate(
            [ei[1], jnp.full((NW, npad_e), N, jnp.int32)], axis=1)
    else:
        src, dst = ei[0], ei[1]
    dst = dst.reshape(NW, NCHUNK, CHUNK)
    zero = jnp.zeros((NPAD, D), jnp.float32)
    agg2 = _make_sc_aggregate()(x, src, dst, zero)
    return _tc_mlp_bn(x, agg2, W1, b1, W2, b2,
                      eps.astype(jnp.float32), gamma, beta)


# speedup vs baseline: 1.3301x; 1.3301x over previous
import functools

import jax
import jax.numpy as jnp
from jax import lax
from jax.experimental import pallas as pl
from jax.experimental.pallas import tpu as pltpu
from jax.experimental.pallas import tpu_sc as plsc

N = 10000
E = 320000
D = 128

NC = 2
NS = 16
NW = NC * NS

CHUNK = 80
EDGES_PER_W = E // NW
NCHUNK = 125
EDGES_PAD_W = NCHUNK * CHUNK
NPAD = 10112
ROWS_PER_TILE = NPAD // NS

@functools.cache
def _make_sc_aggregate():
    mesh = plsc.VectorSubcoreMesh(
        core_axis_name="c", subcore_axis_name="s", num_cores=NC,
        num_subcores=NS)
    return pl.kernel(
        _sc_aggregate_body,
        out_type=jax.ShapeDtypeStruct((NC, NPAD, D), jnp.float32),
        mesh=mesh,
        scratch_types=[
            pltpu.VMEM((EDGES_PAD_W,), jnp.int32),
            pltpu.VMEM((NCHUNK, CHUNK), jnp.int32),
            pltpu.VMEM((CHUNK, D), jnp.float32),
            pltpu.VMEM((CHUNK, D), jnp.float32),
            pltpu.VMEM_SHARED((NPAD, D), jnp.float32),
            pltpu.SemaphoreType.DMA,
            pltpu.SemaphoreType.DMA,
            pltpu.SemaphoreType.DMA,
            pltpu.SemaphoreType.DMA,
        ],
    )


def _sc_aggregate_body(x_hbm, src_hbm, dst_hbm, zero_hbm, out_hbm,
                       src_v, dst_v, rows0, rows1, agg_sh,
                       gsem0, gsem1, ssem0, ssem1):
    c = lax.axis_index("c")
    s = lax.axis_index("s")
    wid = s * NC + c

    pltpu.sync_copy(zero_hbm.at[pl.ds(s * ROWS_PER_TILE, ROWS_PER_TILE)],
                    agg_sh.at[pl.ds(s * ROWS_PER_TILE, ROWS_PER_TILE)])
    pltpu.sync_copy(src_hbm.at[wid], src_v)
    pltpu.sync_copy(dst_hbm.at[wid], dst_v)
    plsc.subcore_barrier()

    def gidx(i):
        return src_v.at[pl.ds(pl.multiple_of(i * CHUNK, 8), CHUNK)]

    pltpu.async_copy(x_hbm.at[gidx(0)], rows0, gsem0)

    def pair(t, carry):
        i0 = 2 * t
        i1 = i0 + 1
        pltpu.make_async_copy(x_hbm.at[gidx(i0)], rows0, gsem0).wait()

        @pl.when(t > 0)
        def _():
            pltpu.make_async_copy(rows1, agg_sh.at[dst_v.at[i1]],
                                  ssem1).wait()

        pltpu.async_copy(x_hbm.at[gidx(i1)], rows1, gsem1)
        pltpu.async_copy(rows0, agg_sh.at[dst_v.at[i0]], ssem0, add=True)
        pltpu.make_async_copy(x_hbm.at[gidx(i1)], rows1, gsem1).wait()
        pltpu.make_async_copy(rows0, agg_sh.at[dst_v.at[i0]], ssem0).wait()

        @pl.when(i0 + 2 < NCHUNK)
        def _():
            pltpu.async_copy(x_hbm.at[gidx(i0 + 2)], rows0, gsem0)

        pltpu.async_copy(rows1, agg_sh.at[dst_v.at[i1]], ssem1, add=True)
        return carry

    lax.fori_loop(0, NCHUNK // 2, pair, 0)
    if NCHUNK % 2:
        pltpu.make_async_copy(x_hbm.at[gidx(NCHUNK - 1)], rows0,
                              gsem0).wait()
        pltpu.sync_copy(rows0, agg_sh.at[dst_v.at[NCHUNK - 1]], add=True)
        pltpu.make_async_copy(rows1, agg_sh.at[dst_v.at[NCHUNK - 2]],
                              ssem1).wait()
    else:
        pltpu.make_async_copy(rows1, agg_sh.at[dst_v.at[NCHUNK - 1]],
                              ssem1).wait()
    plsc.subcore_barrier()

    pltpu.sync_copy(agg_sh.at[pl.ds(s * ROWS_PER_TILE, ROWS_PER_TILE)],
                    out_hbm.at[c, pl.ds(s * ROWS_PER_TILE, ROWS_PER_TILE)])


BN = 2000
NB = N // BN


def _tc_body(eps_sm, x_b, a0_b, a1_b, W1_b, b1_b, W2_b, b2_b, g_b, be_b,
             out_b, h2_sc, sum_sc, sq_sc, scale_sc, off_sc):
    p = pl.program_id(0)
    i = pl.program_id(1)

    @pl.when(p == 0)
    def _():
        m = (1.0 + eps_sm[0]) * x_b[...] + a0_b[0] + a1_b[0]
        h1 = jnp.maximum(
            jnp.dot(m, W1_b[...], preferred_element_type=jnp.float32) + b1_b[...],
            0.0)
        h2 = jnp.dot(h1, W2_b[...], preferred_element_type=jnp.float32) + b2_b[...]

        @pl.when(i == 0)
        def _():
            sum_sc[...] = jnp.zeros_like(sum_sc)
            sq_sc[...] = jnp.zeros_like(sq_sc)

        sum_sc[...] += jnp.sum(h2, axis=0, keepdims=True)
        sq_sc[...] += jnp.sum(h2 * h2, axis=0, keepdims=True)
        h2_sc[pl.ds(i * BN, BN), :] = h2

    @pl.when(p == 1)
    def _():
        @pl.when(i == 0)
        def _():
            mean = sum_sc[...] * (1.0 / N)
            var = sq_sc[...] * (1.0 / N) - mean * mean
            sc = lax.rsqrt(var + 1e-5) * g_b[...]
            scale_sc[...] = sc
            off_sc[...] = be_b[...] - mean * sc

        h2 = h2_sc[pl.ds(i * BN, BN), :]
        out_b[...] = jnp.maximum(h2 * scale_sc[...] + off_sc[...], 0.0)


def _row_map(p, i):
    return (jnp.where(p == 0, i, 0), 0)


def _tc_mlp_bn(x, agg2, W1, b1, W2, b2, eps, gamma, beta):
    vec = lambda v: v.reshape(1, D)
    a_map = lambda core: (lambda p, i: (core, jnp.where(p == 0, i, 0), 0))
    return pl.pallas_call(
        _tc_body,
        grid=(2, NB),
        in_specs=[
            pl.BlockSpec(memory_space=pltpu.SMEM),
            pl.BlockSpec((BN, D), _row_map),
            pl.BlockSpec((1, BN, D), a_map(0)),
            pl.BlockSpec((1, BN, D), a_map(1)),
            pl.BlockSpec((D, D), lambda p, i: (0, 0)),
            pl.BlockSpec((1, D), lambda p, i: (0, 0)),
            pl.BlockSpec((D, D), lambda p, i: (0, 0)),
            pl.BlockSpec((1, D), lambda p, i: (0, 0)),
            pl.BlockSpec((1, D), lambda p, i: (0, 0)),
            pl.BlockSpec((1, D), lambda p, i: (0, 0)),
        ],
        out_specs=pl.BlockSpec((BN, D),
                               lambda p, i: (jnp.where(p == 0, 0, i), 0)),
        out_shape=jax.ShapeDtypeStruct((N, D), jnp.float32),
        scratch_shapes=[
            pltpu.VMEM((N, D), jnp.float32),
            pltpu.VMEM((1, D), jnp.float32),
            pltpu.VMEM((1, D), jnp.float32),
            pltpu.VMEM((1, D), jnp.float32),
            pltpu.VMEM((1, D), jnp.float32),
        ],
    )(eps.reshape(1), x, agg2, agg2, W1, vec(b1), W2, vec(b2), vec(gamma),
      vec(beta))


def kernel(x, edge_index, W1, b1, W2, b2, eps, gamma, beta):
    npad_e = EDGES_PAD_W - EDGES_PER_W
    ei = edge_index.reshape(2, NW, EDGES_PER_W)
    if npad_e:
        src = jnp.concatenate(
            [ei[0], jnp.zeros((NW, npad_e), jnp.int32)], axis=1)
        dst = jnp.concatenate(
            [ei[1], jnp.full((NW, npad_e), N, jnp.int32)], axis=1)
    else:
        src, dst = ei[0], ei[1]
    dst = dst.reshape(NW, NCHUNK, CHUNK)
    zero = jnp.zeros((NPAD, D), jnp.float32)
    agg2 = _make_sc_aggregate()(x, src, dst, zero)
    return _tc_mlp_bn(x, agg2, W1, b1, W2, b2,
                      eps.astype(jnp.float32), gamma, beta)
